# baseline (speedup 1.0000x reference)
import functools

import jax
import jax.numpy as jnp
from jax import lax
from jax.experimental import pallas as pl
from jax.experimental.pallas import tpu as pltpu
from jax.experimental.pallas import tpu_sc as plsc

_BATCH = 4096
_CTX = 50
_D = 128
_H = 512
_C = 1000

_NC, _NS = 2, 16
_NW = _NC * _NS
_BPW = _BATCH // _NW


@functools.cache
def _make_bag_kernel():
    @functools.partial(
        pl.kernel,
        out_type=jax.ShapeDtypeStruct((_BATCH, _D), jnp.float32),
        mesh=plsc.VectorSubcoreMesh(core_axis_name="c", subcore_axis_name="s",
                                    num_cores=_NC, num_subcores=_NS),
        scratch_types=[
            pltpu.VMEM((_CTX, _BPW), jnp.int32),
            pltpu.VMEM((_BPW, _D), jnp.float32),
            pltpu.SemaphoreType.DMA,
        ],
    )
    def _bag_kernel(idx_hbm, emb_hbm, out_hbm, idx_v, acc_v, sem):
        c = lax.axis_index("c")
        s = lax.axis_index("s")
        wid = s * _NC + c
        pltpu.sync_copy(idx_hbm.at[wid], idx_v)

        @pl.loop(0, _BPW)
        def _(i):
            @pl.loop(0, _D, step=16)
            def _(j):
                acc_v[i, pl.ds(j, 16)] = jnp.zeros((16,), jnp.float32)

        @pl.loop(0, 16)
        def _(t):
            pltpu.async_copy(emb_hbm.at[idx_v.at[t]], acc_v, sem, add=True)

        @pl.loop(16, _CTX)
        def _(t):
            pltpu.make_async_copy(emb_hbm.at[idx_v.at[0]], acc_v, sem).wait()
            pltpu.async_copy(emb_hbm.at[idx_v.at[t]], acc_v, sem, add=True)

        @pl.loop(0, 16)
        def _(t):
            pltpu.make_async_copy(emb_hbm.at[idx_v.at[0]], acc_v, sem).wait()

        pltpu.sync_copy(acc_v, out_hbm.at[pl.ds(wid * _BPW, _BPW)])

    return _bag_kernel


_BB = 512


def _mlp_body(bag_ref, w1_ref, b1_ref, w2_ref, b2_ref, out_ref):
    bag = bag_ref[...].astype(jnp.bfloat16)
    h = jnp.dot(bag, w1_ref[...], preferred_element_type=jnp.float32)
    h = jnp.maximum(h + b1_ref[...], 0.0).astype(jnp.bfloat16)
    logits = jnp.dot(h, w2_ref[...], preferred_element_type=jnp.float32) + b2_ref[...]
    m = jnp.max(logits, axis=-1, keepdims=True)
    lse = jnp.log(jnp.sum(jnp.exp(logits - m), axis=-1, keepdims=True)) + m
    out_ref[...] = logits - lse


_mlp = pl.pallas_call(
    _mlp_body,
    grid=(_BATCH // _BB,),
    in_specs=[
        pl.BlockSpec((_BB, _D), lambda i: (i, 0)),
        pl.BlockSpec((_D, _H), lambda i: (0, 0)),
        pl.BlockSpec((1, _H), lambda i: (0, 0)),
        pl.BlockSpec((_H, _C), lambda i: (0, 0)),
        pl.BlockSpec((1, _C), lambda i: (0, 0)),
    ],
    out_specs=pl.BlockSpec((_BB, _C), lambda i: (i, 0)),
    out_shape=jax.ShapeDtypeStruct((_BATCH, _C), jnp.float32),
)


def kernel(indices, emb, W1, b1, W2, b2):
    idx3 = indices.astype(jnp.int32).reshape(_NW, _BPW, _CTX).transpose(0, 2, 1)
    bag = _make_bag_kernel()(idx3, emb)
    return _mlp(bag, W1.astype(jnp.bfloat16), b1.reshape(1, _H),
                W2.astype(jnp.bfloat16), b2.reshape(1, _C))
